# baseline (speedup 1.0000x reference)
import functools

import jax
import jax.numpy as jnp
from jax import lax
from jax.experimental import pallas as pl
from jax.experimental.pallas import tpu as pltpu
from jax.experimental.pallas import tpu_sc as plsc

B = 64
M = 4096
W = 64
_CHUNK = 128
_NCHUNK = M // _CHUNK
_LANES = 16
_MB = 512


def _uw_body(u_ref, uw_ref):
    u = u_ref[...]
    lg = jnp.log(jnp.maximum(u, 1e-37))
    x3 = lg.reshape(B * _NCHUNK, _CHUNK)
    r = lax.broadcasted_iota(jnp.int32, (_CHUNK, _CHUNK), 0)
    c = lax.broadcasted_iota(jnp.int32, (_CHUNK, _CHUNK), 1)
    t_strict = (r < c).astype(jnp.float32)
    e3 = lax.dot(x3, t_strict, precision=lax.Precision.HIGHEST)
    tsum = jnp.sum(lg.reshape(B, _NCHUNK, _CHUNK), axis=2)
    r2 = lax.broadcasted_iota(jnp.int32, (_NCHUNK, _NCHUNK), 0)
    c2 = lax.broadcasted_iota(jnp.int32, (_NCHUNK, _NCHUNK), 1)
    t2_strict = (r2 < c2).astype(jnp.float32)
    p = lax.dot(tsum, t2_strict, precision=lax.Precision.HIGHEST)
    e = e3.reshape(B, _NCHUNK, _CHUNK) + p[:, :, None]
    uw_ref[...] = (1.0 - u) * jnp.exp(e.reshape(B, M))


def _compute_uw(sorted_usage):
    return pl.pallas_call(
        _uw_body,
        out_shape=jax.ShapeDtypeStruct((B, M), jnp.float32),
    )(sorted_usage)


def _scatter_body(uw_hbm, fl_hbm, aw_hbm, idx_v, val_v, aw_v):
    nc = 2
    wid = lax.axis_index("s") * nc + lax.axis_index("c")
    zeros16 = jnp.zeros((_LANES,), jnp.float32)
    n_per_w = B // 32

    for j in range(n_per_w):
        b = wid * n_per_w + j
        pltpu.sync_copy(fl_hbm.at[b], idx_v)
        pltpu.sync_copy(uw_hbm.at[b], val_v)

        def zero_body(k, carry):
            aw_v[pl.ds(k * _LANES, _LANES)] = zeros16
            return carry

        lax.fori_loop(0, M // _LANES, zero_body, 0)

        def scat_body(k, carry):
            iv = idx_v[pl.ds(k * _LANES, _LANES)]
            vv = val_v[pl.ds(k * _LANES, _LANES)]
            plsc.store_scatter(aw_v, [iv], vv)
            return carry

        lax.fori_loop(0, M // _LANES, scat_body, 0)
        pltpu.sync_copy(aw_v, aw_hbm.at[b])


@functools.cache
def _scatter_sc():
    return pl.kernel(
        _scatter_body,
        out_type=jax.ShapeDtypeStruct((B, M), jnp.float32),
        mesh=plsc.VectorSubcoreMesh(
            core_axis_name="c", subcore_axis_name="s", num_cores=2, num_subcores=16
        ),
        scratch_types=[
            pltpu.VMEM((M,), jnp.int32),
            pltpu.VMEM((M,), jnp.float32),
            pltpu.VMEM((M,), jnp.float32),
        ],
        compiler_params=pltpu.CompilerParams(needs_layout_passes=False),
    )


def _update_body(mem_ref, aw_ref, wv_ref, ev_ref, out_ref):
    m = mem_ref[0]
    a = aw_ref[0]
    w = wv_ref[0]
    e = ev_ref[0]
    out_ref[0] = m * (1.0 - a * e) + a * w


def _update_memory(mem_t, aw2, wv2, ev2):
    grid = (B,)
    return pl.pallas_call(
        _update_body,
        grid=grid,
        in_specs=[
            pl.BlockSpec((1, W, M), lambda b: (b, 0, 0)),
            pl.BlockSpec((1, 1, M), lambda b: (b, 0, 0)),
            pl.BlockSpec((1, W, 1), lambda b: (b, 0, 0)),
            pl.BlockSpec((1, W, 1), lambda b: (b, 0, 0)),
        ],
        out_specs=pl.BlockSpec((1, W, M), lambda b: (b, 0, 0)),
        out_shape=jax.ShapeDtypeStruct((B, W, M), jnp.float32),
    )(mem_t, aw2, wv2, ev2)


def kernel(memory_matrix, sorted_usage, write_vector, erase_vector, free_list):
    uw = _compute_uw(sorted_usage)
    aw = _scatter_sc()(uw, free_list)
    mem_t = jnp.swapaxes(memory_matrix, 1, 2)
    out_t = _update_memory(
        mem_t,
        aw.reshape(B, 1, M),
        write_vector.reshape(B, W, 1),
        erase_vector.reshape(B, W, 1),
    )
    return jnp.swapaxes(out_t, 1, 2)


# speedup vs baseline: 1.2287x; 1.2287x over previous
import functools

import jax
import jax.numpy as jnp
from jax import lax
from jax.experimental import pallas as pl
from jax.experimental.pallas import tpu as pltpu
from jax.experimental.pallas import tpu_sc as plsc

B = 64
M = 4096
W = 64
_CHUNK = 128
_NCHUNK = M // _CHUNK
_LANES = 16
_MB = 512


def _uw_body(u_ref, uw_ref):
    u = u_ref[...]
    lg = jnp.log(jnp.maximum(u, 1e-37))
    x3 = lg.reshape(B * _NCHUNK, _CHUNK)
    r = lax.broadcasted_iota(jnp.int32, (_CHUNK, _CHUNK), 0)
    c = lax.broadcasted_iota(jnp.int32, (_CHUNK, _CHUNK), 1)
    t_strict = (r < c).astype(jnp.float32)
    e3 = lax.dot(x3, t_strict, precision=lax.Precision.HIGHEST)
    tsum = jnp.sum(lg.reshape(B, _NCHUNK, _CHUNK), axis=2)
    r2 = lax.broadcasted_iota(jnp.int32, (_NCHUNK, _NCHUNK), 0)
    c2 = lax.broadcasted_iota(jnp.int32, (_NCHUNK, _NCHUNK), 1)
    t2_strict = (r2 < c2).astype(jnp.float32)
    p = lax.dot(tsum, t2_strict, precision=lax.Precision.HIGHEST)
    e = e3.reshape(B, _NCHUNK, _CHUNK) + p[:, :, None]
    uw_ref[...] = (1.0 - u) * jnp.exp(e.reshape(B, M))


def _compute_uw(sorted_usage):
    return pl.pallas_call(
        _uw_body,
        out_shape=jax.ShapeDtypeStruct((B, M), jnp.float32),
    )(sorted_usage)


def _scatter_body(uw_hbm, fl_hbm, aw_hbm, idx_v, val_v, aw_v):
    nc = 2
    wid = lax.axis_index("s") * nc + lax.axis_index("c")
    zeros16 = jnp.zeros((_LANES,), jnp.float32)
    n_per_w = B // 32

    for j in range(n_per_w):
        b = wid * n_per_w + j
        pltpu.sync_copy(fl_hbm.at[b], idx_v)
        pltpu.sync_copy(uw_hbm.at[b], val_v)

        def zero_body(k, carry):
            aw_v[pl.ds(k * _LANES, _LANES)] = zeros16
            return carry

        lax.fori_loop(0, M // _LANES, zero_body, 0)

        def scat_body(k, carry):
            iv = idx_v[pl.ds(k * _LANES, _LANES)]
            vv = val_v[pl.ds(k * _LANES, _LANES)]
            plsc.store_scatter(aw_v, [iv], vv)
            return carry

        lax.fori_loop(0, M // _LANES, scat_body, 0)
        pltpu.sync_copy(aw_v, aw_hbm.at[b])


@functools.cache
def _scatter_sc():
    return pl.kernel(
        _scatter_body,
        out_type=jax.ShapeDtypeStruct((B, M), jnp.float32),
        mesh=plsc.VectorSubcoreMesh(
            core_axis_name="c", subcore_axis_name="s", num_cores=2, num_subcores=16
        ),
        scratch_types=[
            pltpu.VMEM((M,), jnp.int32),
            pltpu.VMEM((M,), jnp.float32),
            pltpu.VMEM((M,), jnp.float32),
        ],
        compiler_params=pltpu.CompilerParams(needs_layout_passes=False),
    )


def _update_body(mem_ref, aw_ref, wv_ref, ev_ref, out_ref):
    m = mem_ref[...]
    a = aw_ref[...]
    w = wv_ref[...]
    e = ev_ref[...]
    out_ref[...] = m * (1.0 - a * e) + a * w


def _update_memory(mem_t, aw2, wv2, ev2):
    grid = (B // 2,)
    return pl.pallas_call(
        _update_body,
        grid=grid,
        in_specs=[
            pl.BlockSpec((2, W, M), lambda b: (b, 0, 0)),
            pl.BlockSpec((2, 1, M), lambda b: (b, 0, 0)),
            pl.BlockSpec((2, W, 1), lambda b: (b, 0, 0)),
            pl.BlockSpec((2, W, 1), lambda b: (b, 0, 0)),
        ],
        out_specs=pl.BlockSpec((2, W, M), lambda b: (b, 0, 0)),
        out_shape=jax.ShapeDtypeStruct((B, W, M), jnp.float32),
    )(mem_t, aw2, wv2, ev2)


def kernel(memory_matrix, sorted_usage, write_vector, erase_vector, free_list):
    uw = _compute_uw(sorted_usage)
    aw = _scatter_sc()(uw, free_list)
    mem_t = jnp.swapaxes(memory_matrix, 1, 2)
    out_t = _update_memory(
        mem_t,
        aw.reshape(B, 1, M),
        write_vector.reshape(B, W, 1),
        erase_vector.reshape(B, W, 1),
    )
    return jnp.swapaxes(out_t, 1, 2)


# speedup vs baseline: 1.3238x; 1.0773x over previous
import functools

import jax
import jax.numpy as jnp
from jax import lax
from jax.experimental import pallas as pl
from jax.experimental.pallas import tpu as pltpu
from jax.experimental.pallas import tpu_sc as plsc

B = 64
M = 4096
W = 64
_CHUNK = 128
_NCHUNK = M // _CHUNK
_LANES = 16
_MB = 512


def _uw_body(u_ref, uw_ref):
    u = u_ref[...]
    lg = jnp.log(jnp.maximum(u, 1e-37))
    x3 = lg.reshape(B * _NCHUNK, _CHUNK)
    r = lax.broadcasted_iota(jnp.int32, (_CHUNK, _CHUNK), 0)
    c = lax.broadcasted_iota(jnp.int32, (_CHUNK, _CHUNK), 1)
    t_strict = (r < c).astype(jnp.float32)
    e3 = lax.dot(x3, t_strict, precision=lax.Precision.HIGHEST)
    tsum = jnp.sum(lg.reshape(B, _NCHUNK, _CHUNK), axis=2)
    r2 = lax.broadcasted_iota(jnp.int32, (_NCHUNK, _NCHUNK), 0)
    c2 = lax.broadcasted_iota(jnp.int32, (_NCHUNK, _NCHUNK), 1)
    t2_strict = (r2 < c2).astype(jnp.float32)
    p = lax.dot(tsum, t2_strict, precision=lax.Precision.HIGHEST)
    e = e3.reshape(B, _NCHUNK, _CHUNK) + p[:, :, None]
    uw_ref[...] = (1.0 - u) * jnp.exp(e.reshape(B, M))


def _compute_uw(sorted_usage):
    return pl.pallas_call(
        _uw_body,
        out_shape=jax.ShapeDtypeStruct((B, M), jnp.float32),
    )(sorted_usage)


def _scatter_body(uw_hbm, fl_hbm, aw_hbm, idx_v, val_v, aw_v):
    nc = 2
    wid = lax.axis_index("s") * nc + lax.axis_index("c")
    zeros16 = jnp.zeros((_LANES,), jnp.float32)
    n_per_w = B // 32

    for j in range(n_per_w):
        b = wid * n_per_w + j
        pltpu.sync_copy(fl_hbm.at[b], idx_v)
        pltpu.sync_copy(uw_hbm.at[b], val_v)

        def zero_body(k, carry):
            aw_v[pl.ds(k * _LANES, _LANES)] = zeros16
            return carry

        lax.fori_loop(0, M // _LANES, zero_body, 0)

        def scat_body(k, carry):
            iv = idx_v[pl.ds(k * _LANES, _LANES)]
            vv = val_v[pl.ds(k * _LANES, _LANES)]
            plsc.store_scatter(aw_v, [iv], vv)
            return carry

        lax.fori_loop(0, M // _LANES, scat_body, 0)
        pltpu.sync_copy(aw_v, aw_hbm.at[b])


@functools.cache
def _scatter_sc():
    return pl.kernel(
        _scatter_body,
        out_type=jax.ShapeDtypeStruct((B, M), jnp.float32),
        mesh=plsc.VectorSubcoreMesh(
            core_axis_name="c", subcore_axis_name="s", num_cores=2, num_subcores=16
        ),
        scratch_types=[
            pltpu.VMEM((M,), jnp.int32),
            pltpu.VMEM((M,), jnp.float32),
            pltpu.VMEM((M,), jnp.float32),
        ],
        compiler_params=pltpu.CompilerParams(needs_layout_passes=False),
    )


def _update_body(mem_ref, aw_ref, wv_ref, ev_ref, out_ref):
    m = mem_ref[...]
    a = aw_ref[...]
    w = wv_ref[...]
    e = ev_ref[...]
    out_ref[...] = m * (1.0 - a * e) + a * w


def _update_memory(mem_t, aw2, wv2, ev2):
    grid = (B // 4,)
    return pl.pallas_call(
        _update_body,
        grid=grid,
        in_specs=[
            pl.BlockSpec((4, W, M), lambda b: (b, 0, 0)),
            pl.BlockSpec((4, 1, M), lambda b: (b, 0, 0)),
            pl.BlockSpec((4, W, 1), lambda b: (b, 0, 0)),
            pl.BlockSpec((4, W, 1), lambda b: (b, 0, 0)),
        ],
        out_specs=pl.BlockSpec((4, W, M), lambda b: (b, 0, 0)),
        out_shape=jax.ShapeDtypeStruct((B, W, M), jnp.float32),
    )(mem_t, aw2, wv2, ev2)


def kernel(memory_matrix, sorted_usage, write_vector, erase_vector, free_list):
    uw = _compute_uw(sorted_usage)
    aw = _scatter_sc()(uw, free_list)
    mem_t = jnp.swapaxes(memory_matrix, 1, 2)
    out_t = _update_memory(
        mem_t,
        aw.reshape(B, 1, M),
        write_vector.reshape(B, W, 1),
        erase_vector.reshape(B, W, 1),
    )
    return jnp.swapaxes(out_t, 1, 2)


# speedup vs baseline: 1.3402x; 1.0124x over previous
import functools

import jax
import jax.numpy as jnp
from jax import lax
from jax.experimental import pallas as pl
from jax.experimental.pallas import tpu as pltpu
from jax.experimental.pallas import tpu_sc as plsc

B = 64
M = 4096
W = 64
_CHUNK = 128
_NCHUNK = M // _CHUNK
_LANES = 16
_MB = 512


def _uw_body(u_ref, uw_ref):
    u = u_ref[...]
    lg = jnp.log(jnp.maximum(u, 1e-37))
    x3 = lg.reshape(B * _NCHUNK, _CHUNK)
    r = lax.broadcasted_iota(jnp.int32, (_CHUNK, _CHUNK), 0)
    c = lax.broadcasted_iota(jnp.int32, (_CHUNK, _CHUNK), 1)
    t_strict = (r < c).astype(jnp.float32)
    e3 = lax.dot(x3, t_strict, precision=lax.Precision.HIGHEST)
    tsum = jnp.sum(lg.reshape(B, _NCHUNK, _CHUNK), axis=2)
    r2 = lax.broadcasted_iota(jnp.int32, (_NCHUNK, _NCHUNK), 0)
    c2 = lax.broadcasted_iota(jnp.int32, (_NCHUNK, _NCHUNK), 1)
    t2_strict = (r2 < c2).astype(jnp.float32)
    p = lax.dot(tsum, t2_strict, precision=lax.Precision.HIGHEST)
    e = e3.reshape(B, _NCHUNK, _CHUNK) + p[:, :, None]
    uw_ref[...] = (1.0 - u) * jnp.exp(e.reshape(B, M))


def _compute_uw(sorted_usage):
    return pl.pallas_call(
        _uw_body,
        out_shape=jax.ShapeDtypeStruct((B, M), jnp.float32),
    )(sorted_usage)


def _scatter_body(uw_hbm, fl_hbm, aw_hbm, idx_v, val_v, aw_v):
    nc = 2
    wid = lax.axis_index("s") * nc + lax.axis_index("c")
    zeros16 = jnp.zeros((_LANES,), jnp.float32)
    n_per_w = B // 32

    for j in range(n_per_w):
        b = wid * n_per_w + j
        pltpu.sync_copy(fl_hbm.at[b], idx_v)
        pltpu.sync_copy(uw_hbm.at[b], val_v)

        def zero_body(k, carry):
            aw_v[pl.ds(k * _LANES, _LANES)] = zeros16
            return carry

        lax.fori_loop(0, M // _LANES, zero_body, 0)

        def scat_body(k, carry):
            iv = idx_v[pl.ds(k * _LANES, _LANES)]
            vv = val_v[pl.ds(k * _LANES, _LANES)]
            plsc.store_scatter(aw_v, [iv], vv)
            return carry

        lax.fori_loop(0, M // _LANES, scat_body, 0)
        pltpu.sync_copy(aw_v, aw_hbm.at[b])


@functools.cache
def _scatter_sc():
    return pl.kernel(
        _scatter_body,
        out_type=jax.ShapeDtypeStruct((B, M), jnp.float32),
        mesh=plsc.VectorSubcoreMesh(
            core_axis_name="c", subcore_axis_name="s", num_cores=2, num_subcores=16
        ),
        scratch_types=[
            pltpu.VMEM((M,), jnp.int32),
            pltpu.VMEM((M,), jnp.float32),
            pltpu.VMEM((M,), jnp.float32),
        ],
        compiler_params=pltpu.CompilerParams(needs_layout_passes=False),
    )


def _update_body(mem_ref, aw_ref, wv_ref, ev_ref, out_ref):
    m = mem_ref[...]
    a = aw_ref[...]
    w = wv_ref[...]
    e = ev_ref[...]
    out_ref[...] = m * (1.0 - a * e) + a * w


def _update_memory(mem_t, aw2, wv2, ev2):
    grid = (B // 8,)
    return pl.pallas_call(
        _update_body,
        grid=grid,
        in_specs=[
            pl.BlockSpec((8, W, M), lambda b: (b, 0, 0)),
            pl.BlockSpec((8, 1, M), lambda b: (b, 0, 0)),
            pl.BlockSpec((8, W, 1), lambda b: (b, 0, 0)),
            pl.BlockSpec((8, W, 1), lambda b: (b, 0, 0)),
        ],
        out_specs=pl.BlockSpec((8, W, M), lambda b: (b, 0, 0)),
        out_shape=jax.ShapeDtypeStruct((B, W, M), jnp.float32),
    )(mem_t, aw2, wv2, ev2)


def kernel(memory_matrix, sorted_usage, write_vector, erase_vector, free_list):
    uw = _compute_uw(sorted_usage)
    aw = _scatter_sc()(uw, free_list)
    mem_t = jnp.swapaxes(memory_matrix, 1, 2)
    out_t = _update_memory(
        mem_t,
        aw.reshape(B, 1, M),
        write_vector.reshape(B, W, 1),
        erase_vector.reshape(B, W, 1),
    )
    return jnp.swapaxes(out_t, 1, 2)


# speedup vs baseline: 1.3551x; 1.0111x over previous
import functools

import jax
import jax.numpy as jnp
from jax import lax
from jax.experimental import pallas as pl
from jax.experimental.pallas import tpu as pltpu
from jax.experimental.pallas import tpu_sc as plsc

B = 64
M = 4096
W = 64
_CHUNK = 128
_NCHUNK = M // _CHUNK
_LANES = 16
_MB = 512


def _uw_body(u_ref, uw_ref):
    u = u_ref[...]
    lg = jnp.log(jnp.maximum(u, 1e-37))
    x3 = lg.reshape(B * _NCHUNK, _CHUNK)
    r = lax.broadcasted_iota(jnp.int32, (_CHUNK, _CHUNK), 0)
    c = lax.broadcasted_iota(jnp.int32, (_CHUNK, _CHUNK), 1)
    t_strict = (r < c).astype(jnp.float32)
    e3 = lax.dot(x3, t_strict, precision=lax.Precision.HIGHEST)
    tsum = jnp.sum(lg.reshape(B, _NCHUNK, _CHUNK), axis=2)
    r2 = lax.broadcasted_iota(jnp.int32, (_NCHUNK, _NCHUNK), 0)
    c2 = lax.broadcasted_iota(jnp.int32, (_NCHUNK, _NCHUNK), 1)
    t2_strict = (r2 < c2).astype(jnp.float32)
    p = lax.dot(tsum, t2_strict, precision=lax.Precision.HIGHEST)
    e = e3.reshape(B, _NCHUNK, _CHUNK) + p[:, :, None]
    uw_ref[...] = (1.0 - u) * jnp.exp(e.reshape(B, M))


def _compute_uw(sorted_usage):
    return pl.pallas_call(
        _uw_body,
        out_shape=jax.ShapeDtypeStruct((B, M), jnp.float32),
    )(sorted_usage)


_UNROLL = 4


def _scatter_body(uw_hbm, fl_hbm, z_hbm, aw_hbm, idx0, val0, idx1, val1,
                  aw0, aw1, s0, s1, s2, s3, so0, so1):
    nc = 2
    wid = lax.axis_index("s") * nc + lax.axis_index("c")
    b0 = wid * 2
    b1 = b0 + 1
    c_i0 = pltpu.async_copy(fl_hbm.at[b0], idx0, s0)
    c_v0 = pltpu.async_copy(uw_hbm.at[b0], val0, s1)
    c_i1 = pltpu.async_copy(fl_hbm.at[b1], idx1, s2)
    c_v1 = pltpu.async_copy(uw_hbm.at[b1], val1, s3)
    pltpu.sync_copy(z_hbm, aw0)
    pltpu.sync_copy(z_hbm, aw1)

    def do_batch(idx_v, val_v, aw_v):
        def scat_body(k, carry):
            base = k * (_LANES * _UNROLL)
            for u in range(_UNROLL):
                iv = idx_v[pl.ds(base + u * _LANES, _LANES)]
                vv = val_v[pl.ds(base + u * _LANES, _LANES)]
                plsc.store_scatter(aw_v, [iv], vv)
            return carry

        lax.fori_loop(0, M // (_LANES * _UNROLL), scat_body, 0)

    c_i0.wait()
    c_v0.wait()
    do_batch(idx0, val0, aw0)
    o0 = pltpu.async_copy(aw0, aw_hbm.at[b0], so0)
    c_i1.wait()
    c_v1.wait()
    do_batch(idx1, val1, aw1)
    o1 = pltpu.async_copy(aw1, aw_hbm.at[b1], so1)
    o0.wait()
    o1.wait()


@functools.cache
def _scatter_sc():
    return pl.kernel(
        _scatter_body,
        out_type=jax.ShapeDtypeStruct((B, M), jnp.float32),
        mesh=plsc.VectorSubcoreMesh(
            core_axis_name="c", subcore_axis_name="s", num_cores=2, num_subcores=16
        ),
        scratch_types=[
            pltpu.VMEM((M,), jnp.int32),
            pltpu.VMEM((M,), jnp.float32),
            pltpu.VMEM((M,), jnp.int32),
            pltpu.VMEM((M,), jnp.float32),
            pltpu.VMEM((M,), jnp.float32),
            pltpu.VMEM((M,), jnp.float32),
            pltpu.SemaphoreType.DMA,
            pltpu.SemaphoreType.DMA,
            pltpu.SemaphoreType.DMA,
            pltpu.SemaphoreType.DMA,
            pltpu.SemaphoreType.DMA,
            pltpu.SemaphoreType.DMA,
        ],
        compiler_params=pltpu.CompilerParams(needs_layout_passes=False),
    )


def _update_body(mem_ref, aw_ref, wv_ref, ev_ref, out_ref):
    m = mem_ref[...]
    a = aw_ref[...]
    w = wv_ref[...]
    e = ev_ref[...]
    out_ref[...] = m * (1.0 - a * e) + a * w


def _update_memory(mem_t, aw2, wv2, ev2):
    grid = (B // 8,)
    return pl.pallas_call(
        _update_body,
        grid=grid,
        in_specs=[
            pl.BlockSpec((8, W, M), lambda b: (b, 0, 0)),
            pl.BlockSpec((8, 1, M), lambda b: (b, 0, 0)),
            pl.BlockSpec((8, W, 1), lambda b: (b, 0, 0)),
            pl.BlockSpec((8, W, 1), lambda b: (b, 0, 0)),
        ],
        out_specs=pl.BlockSpec((8, W, M), lambda b: (b, 0, 0)),
        out_shape=jax.ShapeDtypeStruct((B, W, M), jnp.float32),
    )(mem_t, aw2, wv2, ev2)


def kernel(memory_matrix, sorted_usage, write_vector, erase_vector, free_list):
    uw = _compute_uw(sorted_usage)
    aw = _scatter_sc()(uw, free_list, jnp.zeros((M,), jnp.float32))
    mem_t = jnp.swapaxes(memory_matrix, 1, 2)
    out_t = _update_memory(
        mem_t,
        aw.reshape(B, 1, M),
        write_vector.reshape(B, W, 1),
        erase_vector.reshape(B, W, 1),
    )
    return jnp.swapaxes(out_t, 1, 2)


# speedup vs baseline: 1.3565x; 1.0011x over previous
import functools

import jax
import jax.numpy as jnp
from jax import lax
from jax.experimental import pallas as pl
from jax.experimental.pallas import tpu as pltpu
from jax.experimental.pallas import tpu_sc as plsc

B = 64
M = 4096
W = 64
_CHUNK = 128
_NCHUNK = M // _CHUNK
_LANES = 16
_MB = 512


def _uw_body(u_hbm, uw_hbm, u_v, uw_v, sem):
    pltpu.async_copy(u_hbm, u_v, sem).wait()
    u = u_v[...]
    lg = jnp.log(jnp.maximum(u, 1e-37))
    x3 = lg.reshape(B * _NCHUNK, _CHUNK)
    r = lax.broadcasted_iota(jnp.int32, (_CHUNK, _CHUNK), 0)
    c = lax.broadcasted_iota(jnp.int32, (_CHUNK, _CHUNK), 1)
    t_strict = (r < c).astype(jnp.float32)
    e3 = lax.dot(x3, t_strict, precision=lax.Precision.HIGHEST)
    tsum = jnp.sum(lg.reshape(B, _NCHUNK, _CHUNK), axis=2)
    r2 = lax.broadcasted_iota(jnp.int32, (_NCHUNK, _NCHUNK), 0)
    c2 = lax.broadcasted_iota(jnp.int32, (_NCHUNK, _NCHUNK), 1)
    t2_strict = (r2 < c2).astype(jnp.float32)
    p = lax.dot(tsum, t2_strict, precision=lax.Precision.HIGHEST)
    e = e3.reshape(B, _NCHUNK, _CHUNK) + p[:, :, None]
    uw_v[...] = (1.0 - u) * jnp.exp(e.reshape(B, M))
    pltpu.async_copy(uw_v, uw_hbm, sem).wait()


def _compute_uw(sorted_usage):
    return pl.pallas_call(
        _uw_body,
        in_specs=[pl.BlockSpec(memory_space=pltpu.MemorySpace.HBM)],
        out_specs=pl.BlockSpec(memory_space=pltpu.MemorySpace.HBM),
        out_shape=jax.ShapeDtypeStruct((B, M), jnp.float32),
        scratch_shapes=[
            pltpu.VMEM((B, M), jnp.float32),
            pltpu.VMEM((B, M), jnp.float32),
            pltpu.SemaphoreType.DMA,
        ],
    )(sorted_usage)


_UNROLL = 4


def _scatter_body(uw_hbm, fl_hbm, z_hbm, aw_hbm, idx0, val0, idx1, val1,
                  aw0, aw1, s0, s1, s2, s3, so0, so1):
    nc = 2
    wid = lax.axis_index("s") * nc + lax.axis_index("c")
    b0 = wid * 2
    b1 = b0 + 1
    c_i0 = pltpu.async_copy(fl_hbm.at[b0], idx0, s0)
    c_v0 = pltpu.async_copy(uw_hbm.at[b0], val0, s1)
    c_i1 = pltpu.async_copy(fl_hbm.at[b1], idx1, s2)
    c_v1 = pltpu.async_copy(uw_hbm.at[b1], val1, s3)
    pltpu.sync_copy(z_hbm, aw0)
    pltpu.sync_copy(z_hbm, aw1)

    def do_batch(idx_v, val_v, aw_v):
        def scat_body(k, carry):
            base = k * (_LANES * _UNROLL)
            for u in range(_UNROLL):
                iv = idx_v[pl.ds(base + u * _LANES, _LANES)]
                vv = val_v[pl.ds(base + u * _LANES, _LANES)]
                plsc.store_scatter(aw_v, [iv], vv)
            return carry

        lax.fori_loop(0, M // (_LANES * _UNROLL), scat_body, 0)

    c_i0.wait()
    c_v0.wait()
    do_batch(idx0, val0, aw0)
    o0 = pltpu.async_copy(aw0, aw_hbm.at[b0], so0)
    c_i1.wait()
    c_v1.wait()
    do_batch(idx1, val1, aw1)
    o1 = pltpu.async_copy(aw1, aw_hbm.at[b1], so1)
    o0.wait()
    o1.wait()


@functools.cache
def _scatter_sc():
    return pl.kernel(
        _scatter_body,
        out_type=jax.ShapeDtypeStruct((B, M), jnp.float32),
        mesh=plsc.VectorSubcoreMesh(
            core_axis_name="c", subcore_axis_name="s", num_cores=2, num_subcores=16
        ),
        scratch_types=[
            pltpu.VMEM((M,), jnp.int32),
            pltpu.VMEM((M,), jnp.float32),
            pltpu.VMEM((M,), jnp.int32),
            pltpu.VMEM((M,), jnp.float32),
            pltpu.VMEM((M,), jnp.float32),
            pltpu.VMEM((M,), jnp.float32),
            pltpu.SemaphoreType.DMA,
            pltpu.SemaphoreType.DMA,
            pltpu.SemaphoreType.DMA,
            pltpu.SemaphoreType.DMA,
            pltpu.SemaphoreType.DMA,
            pltpu.SemaphoreType.DMA,
        ],
        compiler_params=pltpu.CompilerParams(needs_layout_passes=False),
    )


def _update_body(mem_ref, aw_ref, wv_ref, ev_ref, out_ref):
    m = mem_ref[...]
    a = aw_ref[...]
    w = wv_ref[...]
    e = ev_ref[...]
    out_ref[...] = m * (1.0 - a * e) + a * w


def _update_memory(mem_t, aw2, wv2, ev2):
    grid = (B // 8,)
    return pl.pallas_call(
        _update_body,
        grid=grid,
        in_specs=[
            pl.BlockSpec((8, W, M), lambda b: (b, 0, 0)),
            pl.BlockSpec((8, 1, M), lambda b: (b, 0, 0)),
            pl.BlockSpec((8, W, 1), lambda b: (b, 0, 0)),
            pl.BlockSpec((8, W, 1), lambda b: (b, 0, 0)),
        ],
        out_specs=pl.BlockSpec((8, W, M), lambda b: (b, 0, 0)),
        out_shape=jax.ShapeDtypeStruct((B, W, M), jnp.float32),
    )(mem_t, aw2, wv2, ev2)


def kernel(memory_matrix, sorted_usage, write_vector, erase_vector, free_list):
    uw = _compute_uw(sorted_usage)
    aw = _scatter_sc()(uw, free_list, jnp.zeros((M,), jnp.float32))
    mem_t = jnp.swapaxes(memory_matrix, 1, 2)
    out_t = _update_memory(
        mem_t,
        aw.reshape(B, 1, M),
        write_vector.reshape(B, W, 1),
        erase_vector.reshape(B, W, 1),
    )
    return jnp.swapaxes(out_t, 1, 2)
